# baseline (speedup 1.0000x reference)
import jax
import jax.numpy as jnp
from jax.experimental import pallas as pl
from jax.experimental.pallas import tpu as pltpu
from jax.experimental.pallas import tpu_sc as plsc

DIM = 64
NC = 2
NS = 16
NW = NC * NS
CHUNK = 128
SCALE = 8.0


def _sc_embed(table, ids3):
    n_chunks = ids3.shape[1]
    mesh = plsc.VectorSubcoreMesh(
        core_axis_name="c", subcore_axis_name="s", num_cores=NC, num_subcores=NS
    )

    def body(table_hbm, idx_hbm, out_hbm, idx_v, rows_v, gsem):
        wid = jax.lax.axis_index("s") * NC + jax.lax.axis_index("c")
        pltpu.sync_copy(idx_hbm.at[wid], idx_v)

        @pl.loop(0, n_chunks)
        def _chunk(j):
            pltpu.async_copy(table_hbm.at[idx_v.at[j]], rows_v, gsem).wait()

            @pl.loop(0, CHUNK, unroll=8)
            def _row(r):
                for d in range(DIM // 16):
                    sl = pl.ds(d * 16, 16)
                    rows_v[r, sl] = rows_v[r, sl] * SCALE

            pltpu.sync_copy(rows_v, out_hbm.at[wid, j])

    f = pl.kernel(
        body,
        out_type=jax.ShapeDtypeStruct((NW, n_chunks, CHUNK, DIM), jnp.float32),
        mesh=mesh,
        compiler_params=pltpu.CompilerParams(use_tc_tiling_on_sc=False),
        scratch_types=[
            pltpu.VMEM((n_chunks, CHUNK), jnp.int32),
            pltpu.VMEM((CHUNK, DIM), jnp.float32),
            pltpu.SemaphoreType.DMA,
        ],
    )
    return f(table, ids3)


def kernel(input_ids, table):
    batch, seq = input_ids.shape
    total = batch * seq
    n_chunks = total // (NW * CHUNK)
    ids3 = input_ids.reshape(NW, n_chunks, CHUNK).astype(jnp.int32)
    out = _sc_embed(table, ids3)
    return out.reshape(batch, seq, DIM)


# speedup vs baseline: 1.1590x; 1.1590x over previous
import jax
import jax.numpy as jnp
from jax.experimental import pallas as pl
from jax.experimental.pallas import tpu as pltpu
from jax.experimental.pallas import tpu_sc as plsc

DIM = 64
NC = 2
NS = 16
NW = NC * NS
CHUNK = 128
SCALE = 8.0


def _sc_embed(table, ids3):
    n_chunks = ids3.shape[1]
    mesh = plsc.VectorSubcoreMesh(
        core_axis_name="c", subcore_axis_name="s", num_cores=NC, num_subcores=NS
    )

    NBUF = 4
    LOOKAHEAD = 3
    assert n_chunks % NBUF == 0 and n_chunks > NBUF

    def body(table_hbm, idx_hbm, out_hbm, idx_v, r0, r1, r2, r3, *sems):
        rows = (r0, r1, r2, r3)
        gsem = sems[:NBUF]
        ssem = sems[NBUF:]
        wid = jax.lax.axis_index("s") * NC + jax.lax.axis_index("c")
        pltpu.sync_copy(idx_hbm.at[wid], idx_v)

        for b in range(LOOKAHEAD):
            pltpu.async_copy(table_hbm.at[idx_v.at[b]], rows[b], gsem[b])

        @pl.loop(0, n_chunks // NBUF)
        def _grp(g):
            for b in range(NBUF):
                j = g * NBUF + b
                jn = j + LOOKAHEAD
                bn = (b + LOOKAHEAD) % NBUF

                @pl.when(jn < n_chunks)
                def _pf():
                    @pl.when(jn >= NBUF)
                    def _w():
                        pltpu.make_async_copy(
                            rows[bn], out_hbm.at[wid, jn - NBUF], ssem[bn]
                        ).wait()

                    pltpu.async_copy(table_hbm.at[idx_v.at[jn]], rows[bn], gsem[bn])

                pltpu.make_async_copy(
                    table_hbm.at[idx_v.at[j]], rows[b], gsem[b]
                ).wait()

                @pl.loop(0, CHUNK, unroll=8)
                def _row(r):
                    for d in range(DIM // 16):
                        sl = pl.ds(d * 16, 16)
                        rows[b][r, sl] = rows[b][r, sl] * SCALE

                pltpu.async_copy(rows[b], out_hbm.at[wid, j], ssem[b])

        for b in range(NBUF):
            pltpu.make_async_copy(
                rows[b], out_hbm.at[wid, n_chunks - NBUF + b], ssem[b]
            ).wait()

    f = pl.kernel(
        body,
        out_type=jax.ShapeDtypeStruct((NW, n_chunks, CHUNK, DIM), jnp.float32),
        mesh=mesh,
        compiler_params=pltpu.CompilerParams(use_tc_tiling_on_sc=False),
        scratch_types=[
            pltpu.VMEM((n_chunks, CHUNK), jnp.int32),
        ]
        + [pltpu.VMEM((CHUNK, DIM), jnp.float32) for _ in range(NBUF)]
        + [pltpu.SemaphoreType.DMA for _ in range(2 * NBUF)],
    )
    return f(table, ids3)


def kernel(input_ids, table):
    batch, seq = input_ids.shape
    total = batch * seq
    n_chunks = total // (NW * CHUNK)
    ids3 = input_ids.reshape(NW, n_chunks, CHUNK).astype(jnp.int32)
    out = _sc_embed(table, ids3)
    return out.reshape(batch, seq, DIM)
